# Initial kernel scaffold; baseline (speedup 1.0000x reference)
import jax
import jax.numpy as jnp
from jax.experimental import pallas as pl


def kernel(x):
    raise NotImplementedError("write your pallas kernel here")


# speedup vs baseline: 1.6944x; 1.6944x over previous
import functools

import jax
import jax.numpy as jnp
from jax.experimental import pallas as pl

_ROWS = 128
_COLS = 32768
_K = 8
_BLOCK_ROWS = 8
_CHUNKS = 256
_NEG = float("-inf")


def _topk_body(x_ref, vals_ref, idx_ref):
    x = x_ref[...]
    m = jnp.max(x, axis=1, keepdims=True)
    s = jnp.sum(jnp.exp(x - m), axis=1, keepdims=True)

    xm = x.reshape(_BLOCK_ROWS, _CHUNKS, 128)
    gidx = (jax.lax.broadcasted_iota(jnp.int32, (1, _CHUNKS, 128), 1) * 128
            + jax.lax.broadcasted_iota(jnp.int32, (1, _CHUNKS, 128), 2))
    lane = jax.lax.broadcasted_iota(jnp.int32, (_BLOCK_ROWS, 128), 1)
    big = jnp.int32(2**30)

    M = jnp.max(xm, axis=1)

    vals = []
    idxs = []
    for _ in range(_K):
        mk = jnp.max(M, axis=1, keepdims=True)
        hit = xm == mk[:, :, None]
        a = jnp.min(
            jnp.where(hit,
                      jax.lax.broadcasted_iota(jnp.int32, (1, _CHUNKS, 128), 1),
                      big),
            axis=1)
        cand = jnp.where(a < _CHUNKS, a * 128 + lane, big)
        ik = jnp.min(cand, axis=1, keepdims=True)
        vals.append(mk)
        idxs.append(ik)
        xm = jnp.where(gidx == ik[:, :, None], _NEG, xm)
        M = jnp.max(xm, axis=1)

    v = jnp.concatenate(vals, axis=1)
    i = jnp.concatenate(idxs, axis=1)
    vals_ref[...] = jnp.exp(v - m) / s
    idx_ref[...] = i


@jax.jit
def kernel(x):
    grid = (_ROWS // _BLOCK_ROWS,)
    v, i = pl.pallas_call(
        _topk_body,
        grid=grid,
        in_specs=[pl.BlockSpec((_BLOCK_ROWS, _COLS), lambda i: (i, 0))],
        out_specs=[
            pl.BlockSpec((_BLOCK_ROWS, _K), lambda i: (i, 0)),
            pl.BlockSpec((_BLOCK_ROWS, _K), lambda i: (i, 0)),
        ],
        out_shape=[
            jax.ShapeDtypeStruct((_ROWS, _K), jnp.float32),
            jax.ShapeDtypeStruct((_ROWS, _K), jnp.int32),
        ],
    )(x)
    return v, i
